# baseline (speedup 1.0000x reference)
import jax
import jax.numpy as jnp
from jax.experimental import pallas as pl
from jax.experimental.pallas import tpu as pltpu

_NB = {0: [0, 1, 12, 16], 1: [1, 0, 20], 2: [2, 20, 3], 3: [3, 2],
       4: [4, 20, 5], 5: [5, 4, 6], 6: [6, 5, 7], 7: [7, 6, 22],
       8: [8, 20, 9], 9: [9, 8, 10], 10: [10, 9, 11], 11: [11, 10, 24],
       12: [12, 0, 13], 13: [13, 12, 14], 14: [14, 13, 15], 15: [15, 14],
       16: [16, 0, 17], 17: [17, 16, 18], 18: [18, 17, 19], 19: [19, 18],
       20: [20, 1, 2, 4, 8], 21: [21, 22], 22: [22, 21, 7], 23: [23, 24],
       24: [24, 23, 11]}
_E = 8
_D = 32
_T = 9
_HID = 128
_J = 25
_TD = _T * _D
_KEYS = ('W_enc', 'b_enc', 'W_mu', 'b_mu', 'W_lv', 'b_lv',
         'W_dec1', 'b_dec1', 'W_dec2', 'b_dec2')
_NBUF = 8


def _moe_kernel(xT_ref, *refs):
    wrefs = refs[:10 * _J * _E]
    (mu_o, lv_o, xh_o, idx_o,
     dx_s, wenc_s, benc_s, wmu_s, bmu_s, wlv_s, blv_s,
     wd1_s, bd1_s, wd2_s, bd2_s, sems) = refs[10 * _J * _E:]
    B = xT_ref.shape[1]

    def expert_copies(i):
        j, e = divmod(i, _E)
        L = len(_NB[j])
        p = i % _NBUF
        src = wrefs[i * 10:(i + 1) * 10]
        dsts = (wenc_s.at[p, 0:_TD * L], benc_s.at[p], wmu_s.at[p],
                bmu_s.at[p], wlv_s.at[p], blv_s.at[p], wd1_s.at[p],
                bd1_s.at[p], wd2_s.at[p], bd2_s.at[p])
        return [pltpu.make_async_copy(s, d, sems.at[p, k])
                for k, (s, d) in enumerate(zip(src, dsts))]

    for i0 in range(min(_NBUF - 1, _J * _E)):
        for c in expert_copies(i0):
            c.start()

    best = None
    for i in range(_J * _E):
        j, e = divmod(i, _E)
        nb = _NB[j]
        L = len(nb)
        p = i % _NBUF

        if i + _NBUF - 1 < _J * _E:
            for c in expert_copies(i + _NBUF - 1):
                c.start()

        if e == 0:
            for k, srcj in enumerate(nb):
                xk = xT_ref[srcj]
                for t in range(_T):
                    dx_s[:, (t * L + k) * _D:(t * L + k + 1) * _D] = (
                        xk[:, t * _D:(t + 1) * _D])

        for c in expert_copies(i):
            c.wait()

        dx = dx_s[:, :_TD * L]
        h = jnp.dot(dx, wenc_s[p, 0:_TD * L],
                    preferred_element_type=jnp.float32)
        h = jnp.maximum(h + benc_s[p][None, :], 0.0)
        mu = jnp.dot(h, wmu_s[p], preferred_element_type=jnp.float32)
        mu = mu + bmu_s[p][None, :]
        lv = jnp.dot(h, wlv_s[p], preferred_element_type=jnp.float32)
        lv = lv + blv_s[p][None, :]
        hd = jnp.dot(mu, wd1_s[p], preferred_element_type=jnp.float32)
        hd = jnp.maximum(hd + bd1_s[p][None, :], 0.0)
        xh = jnp.dot(hd, wd2_s[p], preferred_element_type=jnp.float32)
        xh = xh + bd2_s[p][None, :]
        diff = xh - xT_ref[j]
        err = jnp.mean(diff * diff, axis=-1, keepdims=True)

        if e == 0:
            best = (err, mu, lv, xh, jnp.zeros((B, 1), jnp.int32))
        else:
            m = err < best[0]
            best = (jnp.where(m, err, best[0]),
                    jnp.where(m, mu, best[1]),
                    jnp.where(m, lv, best[2]),
                    jnp.where(m, xh, best[3]),
                    jnp.where(m, e, best[4]))
        if e == _E - 1:
            mu_o[j] = best[1]
            lv_o[j] = best[2]
            xh_o[j] = best[3]
            idx_o[j] = jnp.broadcast_to(best[4], (B, 8))


def kernel(x, params):
    B = x.shape[0]
    xT = x.transpose(2, 0, 1, 3).reshape(_J, B, _TD)

    args = [xT]
    for j in range(_J):
        for e in range(_E):
            for kkey in _KEYS:
                args.append(params[j][e][kkey])

    hbm = pl.BlockSpec(memory_space=pltpu.MemorySpace.HBM)
    in_specs = [pl.BlockSpec(xT.shape, lambda: (0, 0, 0))]
    in_specs += [hbm] * (len(args) - 1)

    mu_o, lv_o, xh_o, idx_o = pl.pallas_call(
        _moe_kernel,
        grid=(),
        in_specs=in_specs,
        out_specs=[
            pl.BlockSpec((_J, B, _D), lambda: (0, 0, 0)),
            pl.BlockSpec((_J, B, _D), lambda: (0, 0, 0)),
            pl.BlockSpec((_J, B, _TD), lambda: (0, 0, 0)),
            pl.BlockSpec((_J, B, 8), lambda: (0, 0, 0)),
        ],
        out_shape=[
            jax.ShapeDtypeStruct((_J, B, _D), jnp.float32),
            jax.ShapeDtypeStruct((_J, B, _D), jnp.float32),
            jax.ShapeDtypeStruct((_J, B, _TD), jnp.float32),
            jax.ShapeDtypeStruct((_J, B, 8), jnp.int32),
        ],
        scratch_shapes=[
            pltpu.VMEM((B, _TD * 5), jnp.float32),
            pltpu.VMEM((_NBUF, _TD * 5, _HID), jnp.float32),
            pltpu.VMEM((_NBUF, _HID), jnp.float32),
            pltpu.VMEM((_NBUF, _HID, _D), jnp.float32),
            pltpu.VMEM((_NBUF, _D), jnp.float32),
            pltpu.VMEM((_NBUF, _HID, _D), jnp.float32),
            pltpu.VMEM((_NBUF, _D), jnp.float32),
            pltpu.VMEM((_NBUF, _D, _HID), jnp.float32),
            pltpu.VMEM((_NBUF, _HID), jnp.float32),
            pltpu.VMEM((_NBUF, _HID, _TD), jnp.float32),
            pltpu.VMEM((_NBUF, _TD), jnp.float32),
            pltpu.SemaphoreType.DMA((_NBUF, 10)),
        ],
    )(*args)

    out_mu = mu_o.transpose(1, 0, 2)
    out_lv = lv_o.transpose(1, 0, 2)
    out_xh = xh_o.reshape(_J, B, _T, _D).transpose(1, 2, 0, 3)
    out_idx = idx_o[:, :, 0].transpose(1, 0)
    return out_mu, out_lv, out_xh, out_idx


# speedup vs baseline: 1.0007x; 1.0007x over previous
import jax
import jax.numpy as jnp
from jax.experimental import pallas as pl
from jax.experimental.pallas import tpu as pltpu

_NB = {0: [0, 1, 12, 16], 1: [1, 0, 20], 2: [2, 20, 3], 3: [3, 2],
       4: [4, 20, 5], 5: [5, 4, 6], 6: [6, 5, 7], 7: [7, 6, 22],
       8: [8, 20, 9], 9: [9, 8, 10], 10: [10, 9, 11], 11: [11, 10, 24],
       12: [12, 0, 13], 13: [13, 12, 14], 14: [14, 13, 15], 15: [15, 14],
       16: [16, 0, 17], 17: [17, 16, 18], 18: [18, 17, 19], 19: [19, 18],
       20: [20, 1, 2, 4, 8], 21: [21, 22], 22: [22, 21, 7], 23: [23, 24],
       24: [24, 23, 11]}
_E = 8
_D = 32
_T = 9
_HID = 128
_J = 25
_TD = _T * _D
_KEYS = ('W_enc', 'b_enc', 'W_mu', 'b_mu', 'W_lv', 'b_lv',
         'W_dec1', 'b_dec1', 'W_dec2', 'b_dec2')
_NBUF = 16


def _moe_kernel(xT_ref, *refs):
    wrefs = refs[:10 * _J * _E]
    (mu_o, lv_o, xh_o, idx_o,
     dx_s, wenc_s, benc_s, wmu_s, bmu_s, wlv_s, blv_s,
     wd1_s, bd1_s, wd2_s, bd2_s, sems) = refs[10 * _J * _E:]
    B = xT_ref.shape[1]

    def expert_copies(i):
        j, e = divmod(i, _E)
        L = len(_NB[j])
        p = i % _NBUF
        src = wrefs[i * 10:(i + 1) * 10]
        dsts = (wenc_s.at[p, 0:_TD * L], benc_s.at[p], wmu_s.at[p],
                bmu_s.at[p], wlv_s.at[p], blv_s.at[p], wd1_s.at[p],
                bd1_s.at[p], wd2_s.at[p], bd2_s.at[p])
        return [pltpu.make_async_copy(s, d, sems.at[p, k])
                for k, (s, d) in enumerate(zip(src, dsts))]

    for i0 in range(min(_NBUF - 1, _J * _E)):
        for c in expert_copies(i0):
            c.start()

    best = None
    for i in range(_J * _E):
        j, e = divmod(i, _E)
        nb = _NB[j]
        L = len(nb)
        p = i % _NBUF

        if i + _NBUF - 1 < _J * _E:
            for c in expert_copies(i + _NBUF - 1):
                c.start()

        if e == 0:
            for k, srcj in enumerate(nb):
                xk = xT_ref[srcj]
                for t in range(_T):
                    dx_s[:, (t * L + k) * _D:(t * L + k + 1) * _D] = (
                        xk[:, t * _D:(t + 1) * _D])

        for c in expert_copies(i):
            c.wait()

        dx = dx_s[:, :_TD * L]
        h = jnp.dot(dx, wenc_s[p, 0:_TD * L],
                    preferred_element_type=jnp.float32)
        h = jnp.maximum(h + benc_s[p][None, :], 0.0)
        mu = jnp.dot(h, wmu_s[p], preferred_element_type=jnp.float32)
        mu = mu + bmu_s[p][None, :]
        lv = jnp.dot(h, wlv_s[p], preferred_element_type=jnp.float32)
        lv = lv + blv_s[p][None, :]
        hd = jnp.dot(mu, wd1_s[p], preferred_element_type=jnp.float32)
        hd = jnp.maximum(hd + bd1_s[p][None, :], 0.0)
        xh = jnp.dot(hd, wd2_s[p], preferred_element_type=jnp.float32)
        xh = xh + bd2_s[p][None, :]
        diff = xh - xT_ref[j]
        err = jnp.mean(diff * diff, axis=-1, keepdims=True)

        if e == 0:
            best = (err, mu, lv, xh, jnp.zeros((B, 1), jnp.int32))
        else:
            m = err < best[0]
            best = (jnp.where(m, err, best[0]),
                    jnp.where(m, mu, best[1]),
                    jnp.where(m, lv, best[2]),
                    jnp.where(m, xh, best[3]),
                    jnp.where(m, e, best[4]))
        if e == _E - 1:
            mu_o[j] = best[1]
            lv_o[j] = best[2]
            xh_o[j] = best[3]
            idx_o[j] = jnp.broadcast_to(best[4], (B, 8))


def kernel(x, params):
    B = x.shape[0]
    xT = x.transpose(2, 0, 1, 3).reshape(_J, B, _TD)

    args = [xT]
    for j in range(_J):
        for e in range(_E):
            for kkey in _KEYS:
                args.append(params[j][e][kkey])

    hbm = pl.BlockSpec(memory_space=pltpu.MemorySpace.HBM)
    in_specs = [pl.BlockSpec(xT.shape, lambda: (0, 0, 0))]
    in_specs += [hbm] * (len(args) - 1)

    mu_o, lv_o, xh_o, idx_o = pl.pallas_call(
        _moe_kernel,
        grid=(),
        in_specs=in_specs,
        out_specs=[
            pl.BlockSpec((_J, B, _D), lambda: (0, 0, 0)),
            pl.BlockSpec((_J, B, _D), lambda: (0, 0, 0)),
            pl.BlockSpec((_J, B, _TD), lambda: (0, 0, 0)),
            pl.BlockSpec((_J, B, 8), lambda: (0, 0, 0)),
        ],
        out_shape=[
            jax.ShapeDtypeStruct((_J, B, _D), jnp.float32),
            jax.ShapeDtypeStruct((_J, B, _D), jnp.float32),
            jax.ShapeDtypeStruct((_J, B, _TD), jnp.float32),
            jax.ShapeDtypeStruct((_J, B, 8), jnp.int32),
        ],
        scratch_shapes=[
            pltpu.VMEM((B, _TD * 5), jnp.float32),
            pltpu.VMEM((_NBUF, _TD * 5, _HID), jnp.float32),
            pltpu.VMEM((_NBUF, _HID), jnp.float32),
            pltpu.VMEM((_NBUF, _HID, _D), jnp.float32),
            pltpu.VMEM((_NBUF, _D), jnp.float32),
            pltpu.VMEM((_NBUF, _HID, _D), jnp.float32),
            pltpu.VMEM((_NBUF, _D), jnp.float32),
            pltpu.VMEM((_NBUF, _D, _HID), jnp.float32),
            pltpu.VMEM((_NBUF, _HID), jnp.float32),
            pltpu.VMEM((_NBUF, _HID, _TD), jnp.float32),
            pltpu.VMEM((_NBUF, _TD), jnp.float32),
            pltpu.SemaphoreType.DMA((_NBUF, 10)),
        ],
    )(*args)

    out_mu = mu_o.transpose(1, 0, 2)
    out_lv = lv_o.transpose(1, 0, 2)
    out_xh = xh_o.reshape(_J, B, _T, _D).transpose(1, 2, 0, 3)
    out_idx = idx_o[:, :, 0].transpose(1, 0)
    return out_mu, out_lv, out_xh, out_idx
